# baseline (speedup 1.0000x reference)
import functools

import jax
import jax.numpy as jnp
from jax import lax
from jax.experimental import pallas as pl
from jax.experimental.pallas import tpu as pltpu
from jax.experimental.pallas import tpu_sc as plsc

V = 100000
B = 16384
D = 300
HALF = 150
DPACK = 256
NSLAB = 2
NC = 2
NS = 16
NW = NC * NS
B_PER_W = B // NW
CHUNK = 128
N_CHUNKS = B_PER_W // CHUNK

TBM = 1024
TGRID = (V + TBM - 1) // TBM

BM = 512
NB_HALF = B // BM


def _pack(x):
    lo = lax.bitcast_convert_type(x[:, :HALF], jnp.uint32)
    hi = lax.bitcast_convert_type(x[:, HALF:], jnp.uint32)
    rnd = jnp.uint32(0x8000)
    w = ((lo + rnd) >> 16) | ((hi + rnd) & jnp.uint32(0xFFFF0000))
    w = jnp.concatenate(
        [w, jnp.zeros((x.shape[0], DPACK - HALF), jnp.uint32)], axis=1)
    return lax.bitcast_convert_type(w, jnp.float32)


def _slab_src_kernel(ts_ref, we_ref, b_ref, wd_ref, s1_ref, s2_ref,
                     m_scr, c_scr):
    @pl.when(pl.program_id(0) == 0)
    def _():
        m = lax.dot_general(
            we_ref[...], wd_ref[...],
            dimension_numbers=(((0,), (0,)), ((), ())),
            preferred_element_type=jnp.float32,
        )
        m_scr[...] = m.astype(jnp.bfloat16)
        c_scr[...] = jnp.dot(b_ref[...], wd_ref[...],
                             preferred_element_type=jnp.float32)

    z = lax.dot_general(
        ts_ref[...].astype(jnp.bfloat16), m_scr[...],
        dimension_numbers=(((0,), (0,)), ((), ())),
        preferred_element_type=jnp.float32,
    ) + c_scr[...]
    s = _pack(z)
    s1_ref[...] = s[:, :128]
    s2_ref[...] = s[:, 128:]


def _slab_tgt_kernel(tt_ref, t1_ref, t2_ref, i_scr):
    @pl.when(pl.program_id(0) == 0)
    def _():
        i_scr[...] = (
            lax.broadcasted_iota(jnp.int32, (D, D), 0)
            == lax.broadcasted_iota(jnp.int32, (D, D), 1)
        ).astype(jnp.bfloat16)

    z = lax.dot_general(
        tt_ref[...].astype(jnp.bfloat16), i_scr[...],
        dimension_numbers=(((0,), (0,)), ((), ())),
        preferred_element_type=jnp.float32,
    )
    t = _pack(z)
    t1_ref[...] = t[:, :128]
    t2_ref[...] = t[:, 128:]


_SLAB_OUT = dict(
    out_specs=[pl.BlockSpec((TBM, 128), lambda i: (i, 0))] * NSLAB,
    out_shape=[jax.ShapeDtypeStruct((V, 128), jnp.float32)] * NSLAB,
)


def _slabs_src(embT_src, W_enc, b_enc, W_dec):
    return pl.pallas_call(
        _slab_src_kernel,
        grid=(TGRID,),
        in_specs=[
            pl.BlockSpec((D, TBM), lambda i: (0, i)),
            pl.BlockSpec((D, D), lambda i: (0, 0)),
            pl.BlockSpec((1, D), lambda i: (0, 0)),
            pl.BlockSpec((D, D), lambda i: (0, 0)),
        ],
        scratch_shapes=[
            pltpu.VMEM((D, D), jnp.bfloat16),
            pltpu.VMEM((1, D), jnp.float32),
        ],
        **_SLAB_OUT,
    )(embT_src, W_enc, b_enc, W_dec)


def _slabs_tgt(embT_tgt):
    return pl.pallas_call(
        _slab_tgt_kernel,
        grid=(TGRID,),
        in_specs=[pl.BlockSpec((D, TBM), lambda i: (0, i))],
        scratch_shapes=[pltpu.VMEM((D, D), jnp.bfloat16)],
        **_SLAB_OUT,
    )(embT_tgt)


def _sc_gather(ids3, s1, s2):
    mesh = plsc.VectorSubcoreMesh(
        core_axis_name="c", subcore_axis_name="s", num_cores=NC, num_subcores=NS
    )

    @functools.partial(
        pl.kernel,
        out_type=[jax.ShapeDtypeStruct((B, 128), jnp.float32)] * NSLAB,
        mesh=mesh,
        scratch_types=[
            pltpu.VMEM((CHUNK,), jnp.int32),
            pltpu.VMEM((CHUNK, 128), jnp.float32),
            pltpu.VMEM((CHUNK, 128), jnp.float32),
            pltpu.SemaphoreType.DMA,
        ],
    )
    def k(ids_hbm, s1h, s2h, x1, x2, idx_v, r1, r2, sem):
        wid = lax.axis_index("s") * NC + lax.axis_index("c")
        base = wid * B_PER_W
        rbufs = (r1, r2)
        outs = (x1, x2)
        tabs = (s1h, s2h)
        for j in range(N_CHUNKS):
            off = base + j * CHUNK
            pltpu.sync_copy(ids_hbm.at[wid, j], idx_v)
            cps = [pltpu.async_copy(tabs[k_].at[idx_v], rbufs[k_], sem)
                   for k_ in range(NSLAB)]
            for cp in cps:
                cp.wait()
            for k_ in range(NSLAB):
                pltpu.sync_copy(rbufs[k_], outs[k_].at[pl.ds(off, CHUNK)])

    return k(ids3, s1, s2)


def _unpack(x1, x2):
    p = lax.bitcast_convert_type(
        jnp.concatenate([x1, x2], axis=1)[:, :HALF], jnp.uint32)
    lo = lax.bitcast_convert_type(p << 16, jnp.float32)
    hi = lax.bitcast_convert_type(p & jnp.uint32(0xFFFF0000), jnp.float32)
    return jnp.concatenate([lo, hi], axis=1)


def _map_kernel(xs1_ref, xs2_ref, xt1_ref, xt2_ref, out_ref):
    i = pl.program_id(0)

    @pl.when(i < NB_HALF)
    def _():
        out_ref[...] = _unpack(xs1_ref[...], xs2_ref[...])

    @pl.when(i >= NB_HALF)
    def _():
        out_ref[...] = _unpack(xt1_ref[...], xt2_ref[...])


def _tc_map(xs1, xs2, xt1, xt2):
    s_map = lambda i: (jnp.minimum(i, NB_HALF - 1), 0)
    t_map = lambda i: (jnp.maximum(i - NB_HALF, 0), 0)
    return pl.pallas_call(
        _map_kernel,
        grid=(2 * B // BM,),
        in_specs=[
            pl.BlockSpec((BM, 128), s_map),
            pl.BlockSpec((BM, 128), s_map),
            pl.BlockSpec((BM, 128), t_map),
            pl.BlockSpec((BM, 128), t_map),
        ],
        out_specs=pl.BlockSpec((BM, D), lambda i: (i, 0)),
        out_shape=jax.ShapeDtypeStruct((2 * B, D), jnp.float32),
    )(xs1, xs2, xt1, xt2)


def kernel(src_id, tgt_id, emb_src, emb_tgt, W_enc, b_enc, W_dec):
    s1, s2 = _slabs_src(emb_src.T, W_enc, b_enc.reshape(1, D), W_dec)
    t1, t2 = _slabs_tgt(emb_tgt.T)
    ids_s = src_id.astype(jnp.int32).reshape(NW, N_CHUNKS, CHUNK)
    ids_t = tgt_id.astype(jnp.int32).reshape(NW, N_CHUNKS, CHUNK)
    xs1, xs2 = _sc_gather(ids_s, s1, s2)
    xt1, xt2 = _sc_gather(ids_t, t1, t2)
    return _tc_map(xs1, xs2, xt1, xt2)


# speedup vs baseline: 1.2761x; 1.2761x over previous
import functools

import jax
import jax.numpy as jnp
from jax import lax
from jax.experimental import pallas as pl
from jax.experimental.pallas import tpu as pltpu
from jax.experimental.pallas import tpu_sc as plsc

V = 100000
B = 16384
D = 300
HALF = 150
DPACK = 256
NSLAB = 2
NC = 2
NS = 16
NW = NC * NS
B_PER_W = B // NW
CHUNK = 128
N_CHUNKS = B_PER_W // CHUNK

TBM = 2048
TGRID = (V + TBM - 1) // TBM

BM = 512


def _pack(x):
    lo = lax.bitcast_convert_type(x[:, :HALF], jnp.uint32)
    hi = lax.bitcast_convert_type(x[:, HALF:], jnp.uint32)
    rnd = jnp.uint32(0x8000)
    w = ((lo + rnd) >> 16) | ((hi + rnd) & jnp.uint32(0xFFFF0000))
    w = jnp.concatenate(
        [w, jnp.zeros((x.shape[0], DPACK - HALF), jnp.uint32)], axis=1)
    return lax.bitcast_convert_type(w, jnp.float32)


def _slab_kernel(ts_ref, tt_ref, we_ref, b_ref, wd_ref,
                 s1_ref, s2_ref, t1_ref, t2_ref, m_scr, i_scr, c_scr):
    @pl.when(pl.program_id(0) == 0)
    def _():
        m = lax.dot_general(
            we_ref[...], wd_ref[...],
            dimension_numbers=(((0,), (0,)), ((), ())),
            preferred_element_type=jnp.float32,
        )
        m_scr[...] = m.astype(jnp.bfloat16)
        i_scr[...] = (
            lax.broadcasted_iota(jnp.int32, (D, D), 0)
            == lax.broadcasted_iota(jnp.int32, (D, D), 1)
        ).astype(jnp.bfloat16)
        c_scr[...] = jnp.dot(b_ref[...], wd_ref[...],
                             preferred_element_type=jnp.float32)

    zs = lax.dot_general(
        ts_ref[...].astype(jnp.bfloat16), m_scr[...],
        dimension_numbers=(((0,), (0,)), ((), ())),
        preferred_element_type=jnp.float32,
    ) + c_scr[...]
    zt = lax.dot_general(
        tt_ref[...].astype(jnp.bfloat16), i_scr[...],
        dimension_numbers=(((0,), (0,)), ((), ())),
        preferred_element_type=jnp.float32,
    )
    s = _pack(zs)
    t = _pack(zt)
    s1_ref[...] = s[:, :128]
    s2_ref[...] = s[:, 128:]
    t1_ref[...] = t[:, :128]
    t2_ref[...] = t[:, 128:]


def _slabs(embT_src, embT_tgt, W_enc, b_enc, W_dec):
    return pl.pallas_call(
        _slab_kernel,
        grid=(TGRID,),
        in_specs=[
            pl.BlockSpec((D, TBM), lambda i: (0, i)),
            pl.BlockSpec((D, TBM), lambda i: (0, i)),
            pl.BlockSpec((D, D), lambda i: (0, 0)),
            pl.BlockSpec((1, D), lambda i: (0, 0)),
            pl.BlockSpec((D, D), lambda i: (0, 0)),
        ],
        out_specs=[pl.BlockSpec((TBM, 128), lambda i: (i, 0))] * (2 * NSLAB),
        out_shape=[jax.ShapeDtypeStruct((V, 128), jnp.float32)] * (2 * NSLAB),
        scratch_shapes=[
            pltpu.VMEM((D, D), jnp.bfloat16),
            pltpu.VMEM((D, D), jnp.bfloat16),
            pltpu.VMEM((1, D), jnp.float32),
        ],
    )(embT_src, embT_tgt, W_enc, b_enc, W_dec)


def _sc_gather(ids3, s1, s2, t1, t2):
    mesh = plsc.VectorSubcoreMesh(
        core_axis_name="c", subcore_axis_name="s", num_cores=NC, num_subcores=NS
    )

    @functools.partial(
        pl.kernel,
        out_type=[jax.ShapeDtypeStruct((2 * B, 128), jnp.float32)] * NSLAB,
        mesh=mesh,
        scratch_types=[
            pltpu.VMEM((CHUNK,), jnp.int32),
            pltpu.VMEM((CHUNK, 128), jnp.float32),
            pltpu.VMEM((CHUNK, 128), jnp.float32),
            pltpu.SemaphoreType.DMA,
        ],
    )
    def k(ids_hbm, s1h, s2h, t1h, t2h, x1, x2, idx_v, r1, r2, sem):
        wid = lax.axis_index("s") * NC + lax.axis_index("c")
        base = wid * B_PER_W
        rbufs = (r1, r2)
        outs = (x1, x2)
        for half, tabs in enumerate(((s1h, s2h), (t1h, t2h))):
            for j in range(N_CHUNKS):
                off = base + j * CHUNK
                pltpu.sync_copy(ids_hbm.at[half, wid, j], idx_v)
                cps = [pltpu.async_copy(tabs[k_].at[idx_v], rbufs[k_], sem)
                       for k_ in range(NSLAB)]
                for cp in cps:
                    cp.wait()
                dst = half * B + off
                for k_ in range(NSLAB):
                    pltpu.sync_copy(rbufs[k_], outs[k_].at[pl.ds(dst, CHUNK)])

    return k(ids3, s1, s2, t1, t2)


def _unpack(x1, x2):
    p = lax.bitcast_convert_type(
        jnp.concatenate([x1, x2], axis=1)[:, :HALF], jnp.uint32)
    lo = lax.bitcast_convert_type(p << 16, jnp.float32)
    hi = lax.bitcast_convert_type(p & jnp.uint32(0xFFFF0000), jnp.float32)
    return jnp.concatenate([lo, hi], axis=1)


def _map_kernel(x1_ref, x2_ref, out_ref):
    out_ref[...] = _unpack(x1_ref[...], x2_ref[...])


def _tc_map(x1, x2):
    return pl.pallas_call(
        _map_kernel,
        grid=(2 * B // BM,),
        in_specs=[
            pl.BlockSpec((BM, 128), lambda i: (i, 0)),
            pl.BlockSpec((BM, 128), lambda i: (i, 0)),
        ],
        out_specs=pl.BlockSpec((BM, D), lambda i: (i, 0)),
        out_shape=jax.ShapeDtypeStruct((2 * B, D), jnp.float32),
    )(x1, x2)


def kernel(src_id, tgt_id, emb_src, emb_tgt, W_enc, b_enc, W_dec):
    s1, s2, t1, t2 = _slabs(emb_src.T, emb_tgt.T, W_enc, b_enc.reshape(1, D),
                            W_dec)
    ids3 = jnp.stack([src_id.astype(jnp.int32), tgt_id.astype(jnp.int32)]
                     ).reshape(2, NW, N_CHUNKS, CHUNK)
    x1, x2 = _sc_gather(ids3, s1, s2, t1, t2)
    return _tc_map(x1, x2)


# speedup vs baseline: 1.3197x; 1.0342x over previous
import functools

import jax
import jax.numpy as jnp
from jax import lax
from jax.experimental import pallas as pl
from jax.experimental.pallas import tpu as pltpu
from jax.experimental.pallas import tpu_sc as plsc

V = 100000
B = 16384
D = 300
HALF = 150
DPACK = 256
NSLAB = 2
NC = 2
NS = 16
NW = NC * NS
B_PER_W = B // NW
CHUNK = 128
N_CHUNKS = B_PER_W // CHUNK

TBM = 4096
TGRID = (V + TBM - 1) // TBM

BM = 512


def _pack(x):
    lo = lax.bitcast_convert_type(x[:, :HALF], jnp.uint32)
    hi = lax.bitcast_convert_type(x[:, HALF:], jnp.uint32)
    rnd = jnp.uint32(0x8000)
    w = ((lo + rnd) >> 16) | ((hi + rnd) & jnp.uint32(0xFFFF0000))
    w = jnp.concatenate(
        [w, jnp.zeros((x.shape[0], DPACK - HALF), jnp.uint32)], axis=1)
    return lax.bitcast_convert_type(w, jnp.float32)


def _slab_kernel(ts_ref, tt_ref, we_ref, b_ref, wd_ref,
                 s1_ref, s2_ref, t1_ref, t2_ref, m_scr, i_scr, c_scr):
    @pl.when(pl.program_id(0) == 0)
    def _():
        m = lax.dot_general(
            we_ref[...], wd_ref[...],
            dimension_numbers=(((0,), (0,)), ((), ())),
            preferred_element_type=jnp.float32,
        )
        m_scr[...] = m.astype(jnp.bfloat16)
        i_scr[...] = (
            lax.broadcasted_iota(jnp.int32, (D, D), 0)
            == lax.broadcasted_iota(jnp.int32, (D, D), 1)
        ).astype(jnp.bfloat16)
        c_scr[...] = jnp.dot(b_ref[...], wd_ref[...],
                             preferred_element_type=jnp.float32)

    zs = lax.dot_general(
        ts_ref[...].astype(jnp.bfloat16), m_scr[...],
        dimension_numbers=(((0,), (0,)), ((), ())),
        preferred_element_type=jnp.float32,
    ) + c_scr[...]
    zt = lax.dot_general(
        tt_ref[...].astype(jnp.bfloat16), i_scr[...],
        dimension_numbers=(((0,), (0,)), ((), ())),
        preferred_element_type=jnp.float32,
    )
    s = _pack(zs)
    t = _pack(zt)
    s1_ref[...] = s[:, :128]
    s2_ref[...] = s[:, 128:]
    t1_ref[...] = t[:, :128]
    t2_ref[...] = t[:, 128:]


def _slabs(embT_src, embT_tgt, W_enc, b_enc, W_dec):
    return pl.pallas_call(
        _slab_kernel,
        grid=(TGRID,),
        in_specs=[
            pl.BlockSpec((D, TBM), lambda i: (0, i)),
            pl.BlockSpec((D, TBM), lambda i: (0, i)),
            pl.BlockSpec((D, D), lambda i: (0, 0)),
            pl.BlockSpec((1, D), lambda i: (0, 0)),
            pl.BlockSpec((D, D), lambda i: (0, 0)),
        ],
        out_specs=[pl.BlockSpec((TBM, 128), lambda i: (i, 0))] * (2 * NSLAB),
        out_shape=[jax.ShapeDtypeStruct((V, 128), jnp.float32)] * (2 * NSLAB),
        scratch_shapes=[
            pltpu.VMEM((D, D), jnp.bfloat16),
            pltpu.VMEM((D, D), jnp.bfloat16),
            pltpu.VMEM((1, D), jnp.float32),
        ],
    )(embT_src, embT_tgt, W_enc, b_enc, W_dec)


def _sc_gather(ids3, s1, s2, t1, t2):
    mesh = plsc.VectorSubcoreMesh(
        core_axis_name="c", subcore_axis_name="s", num_cores=NC, num_subcores=NS
    )

    @functools.partial(
        pl.kernel,
        out_type=[jax.ShapeDtypeStruct((2 * B, 128), jnp.float32)] * NSLAB,
        mesh=mesh,
        scratch_types=[
            pltpu.VMEM((CHUNK,), jnp.int32),
            pltpu.VMEM((CHUNK, 128), jnp.float32),
            pltpu.VMEM((CHUNK, 128), jnp.float32),
            pltpu.SemaphoreType.DMA,
        ],
    )
    def k(ids_hbm, s1h, s2h, t1h, t2h, x1, x2, idx_v, r1, r2, sem):
        wid = lax.axis_index("s") * NC + lax.axis_index("c")
        base = wid * B_PER_W
        rbufs = (r1, r2)
        outs = (x1, x2)
        for half, tabs in enumerate(((s1h, s2h), (t1h, t2h))):
            for j in range(N_CHUNKS):
                off = base + j * CHUNK
                pltpu.sync_copy(ids_hbm.at[half, wid, j], idx_v)
                cps = [pltpu.async_copy(tabs[k_].at[idx_v], rbufs[k_], sem)
                       for k_ in range(NSLAB)]
                for cp in cps:
                    cp.wait()
                dst = half * B + off
                for k_ in range(NSLAB):
                    pltpu.sync_copy(rbufs[k_], outs[k_].at[pl.ds(dst, CHUNK)])

    return k(ids3, s1, s2, t1, t2)


def _unpack(x1, x2):
    p = lax.bitcast_convert_type(
        jnp.concatenate([x1, x2], axis=1)[:, :HALF], jnp.uint32)
    lo = lax.bitcast_convert_type(p << 16, jnp.float32)
    hi = lax.bitcast_convert_type(p & jnp.uint32(0xFFFF0000), jnp.float32)
    return jnp.concatenate([lo, hi], axis=1)


def _map_kernel(x1_ref, x2_ref, out_ref):
    out_ref[...] = _unpack(x1_ref[...], x2_ref[...])


def _tc_map(x1, x2):
    return pl.pallas_call(
        _map_kernel,
        grid=(2 * B // BM,),
        in_specs=[
            pl.BlockSpec((BM, 128), lambda i: (i, 0)),
            pl.BlockSpec((BM, 128), lambda i: (i, 0)),
        ],
        out_specs=pl.BlockSpec((BM, D), lambda i: (i, 0)),
        out_shape=jax.ShapeDtypeStruct((2 * B, D), jnp.float32),
    )(x1, x2)


def kernel(src_id, tgt_id, emb_src, emb_tgt, W_enc, b_enc, W_dec):
    s1, s2, t1, t2 = _slabs(emb_src.T, emb_tgt.T, W_enc, b_enc.reshape(1, D),
                            W_dec)
    ids3 = jnp.stack([src_id.astype(jnp.int32), tgt_id.astype(jnp.int32)]
                     ).reshape(2, NW, N_CHUNKS, CHUNK)
    x1, x2 = _sc_gather(ids3, s1, s2, t1, t2)
    return _tc_map(x1, x2)
